# Initial kernel scaffold; baseline (speedup 1.0000x reference)
import jax
import jax.numpy as jnp
from jax.experimental import pallas as pl


def kernel(xyz1, xyz2, feats1, feats2, W, gamma, beta):
    raise NotImplementedError("write your pallas kernel here")


# speedup vs baseline: 29.2553x; 29.2553x over previous
import functools

import jax
import jax.numpy as jnp
from jax import lax
from jax.experimental import pallas as pl


def _main_body(x1_ref, x2_ref, f1_ref, f2_ref, wa_ref, wb_ref, y_ref, ss_ref,
               *, tq, n2):
    b = pl.program_id(0)
    n = pl.program_id(1)
    x1 = x1_ref[0]
    x2 = x2_ref[0]
    a0, a1, a2 = x1[:, 0:1], x1[:, 1:2], x1[:, 2:3]
    b0, b1, b2 = x2[0:1, :], x2[1:2, :], x2[2:3, :]
    s1 = a0 * a0 + a1 * a1 + a2 * a2
    s2 = b0 * b0 + b1 * b1 + b2 * b2
    xy = lax.dot_general(x1.astype(jnp.bfloat16), x2.astype(jnp.bfloat16),
                         (((1,), (0,)), ((), ())),
                         preferred_element_type=jnp.float32)
    d2 = s1 + s2 - 2.0 * xy
    dists = jnp.sqrt(jnp.maximum(d2, 1e-12))

    iota = lax.broadcasted_iota(jnp.int32, (tq, n2), 1)
    dcur = dists
    ws = []
    idxs = []
    for k in range(3):
        m = jnp.min(dcur, axis=1, keepdims=True)
        ik = jnp.min(jnp.where(dcur == m, iota, n2), axis=1, keepdims=True)
        ws.append(1.0 / (m + 1e-8))
        idxs.append(ik)
        if k < 2:
            dcur = jnp.where(iota == ik, jnp.inf, dcur)
    wsum = ws[0] + ws[1] + ws[2]

    s_mat = jnp.where(iota == idxs[0], ws[0] / wsum, 0.0)
    s_mat = s_mat + jnp.where(iota == idxs[1], ws[1] / wsum, 0.0)
    s_mat = s_mat + jnp.where(iota == idxs[2], ws[2] / wsum, 0.0)

    f2 = f2_ref[0]
    interp = lax.dot_general(f2, s_mat, (((1,), (1,)), ((), ())))
    y = (jnp.dot(wa_ref[...], interp) +
         jnp.dot(wb_ref[...], f1_ref[0]))
    y_ref[0] = y

    @pl.when((b == 0) & (n == 0))
    def _init():
        ss_ref[...] = jnp.zeros_like(ss_ref)

    part = jnp.concatenate(
        [jnp.sum(y, axis=1, keepdims=True),
         jnp.sum(y * y, axis=1, keepdims=True)], axis=1)
    ss_ref[...] += part


def _bn_body(y_ref, ss_ref, g_ref, bta_ref, o_ref, *, count):
    s = ss_ref[...]
    mean = s[:, 0:1] / count
    var = s[:, 1:2] / count - mean * mean
    scale = g_ref[...] / jnp.sqrt(var + 1e-5)
    shift = bta_ref[...] - mean * scale
    y = y_ref[0]
    o_ref[0] = jnp.maximum(y * scale + shift, 0.0)


def kernel(xyz1, xyz2, feats1, feats2, W, gamma, beta):
    B, N1, _ = xyz1.shape
    N2 = xyz2.shape[1]
    C1 = feats1.shape[1]
    C2 = feats2.shape[1]
    OUT = W.shape[0]
    TQ = 512 if N1 % 512 == 0 else N1

    x1p = jnp.pad(xyz1, ((0, 0), (0, 0), (0, 5)))
    x2p = jnp.pad(jnp.transpose(xyz2, (0, 2, 1)), ((0, 0), (0, 5), (0, 0)))
    Wa = W[:, :C2]
    Wb = W[:, C2:]

    y, ss = pl.pallas_call(
        functools.partial(_main_body, tq=TQ, n2=N2),
        grid=(B, N1 // TQ),
        in_specs=[
            pl.BlockSpec((1, TQ, 8), lambda b, n: (b, n, 0)),
            pl.BlockSpec((1, 8, N2), lambda b, n: (b, 0, 0)),
            pl.BlockSpec((1, C1, TQ), lambda b, n: (b, 0, n)),
            pl.BlockSpec((1, C2, N2), lambda b, n: (b, 0, 0)),
            pl.BlockSpec((OUT, C2), lambda b, n: (0, 0)),
            pl.BlockSpec((OUT, C1), lambda b, n: (0, 0)),
        ],
        out_specs=[
            pl.BlockSpec((1, OUT, TQ), lambda b, n: (b, 0, n)),
            pl.BlockSpec((OUT, 2), lambda b, n: (0, 0)),
        ],
        out_shape=[
            jax.ShapeDtypeStruct((B, OUT, N1), jnp.float32),
            jax.ShapeDtypeStruct((OUT, 2), jnp.float32),
        ],
    )(x1p, x2p, feats1, feats2, Wa, Wb)

    TN = 1024 if N1 % 1024 == 0 else N1
    out = pl.pallas_call(
        functools.partial(_bn_body, count=float(B * N1)),
        grid=(B, N1 // TN),
        in_specs=[
            pl.BlockSpec((1, OUT, TN), lambda b, n: (b, 0, n)),
            pl.BlockSpec((OUT, 2), lambda b, n: (0, 0)),
            pl.BlockSpec((OUT, 1), lambda b, n: (0, 0)),
            pl.BlockSpec((OUT, 1), lambda b, n: (0, 0)),
        ],
        out_specs=pl.BlockSpec((1, OUT, TN), lambda b, n: (b, 0, n)),
        out_shape=jax.ShapeDtypeStruct((B, OUT, N1), jnp.float32),
    )(y, ss, gamma.reshape(OUT, 1), beta.reshape(OUT, 1))
    return out


# speedup vs baseline: 31.9386x; 1.0917x over previous
import functools

import jax
import jax.numpy as jnp
from jax import lax
from jax.experimental import pallas as pl


def _main_body(x1_ref, x2_ref, f1_ref, f2_ref, wa_ref, wb_ref, y_ref, ss_ref,
               *, tq, n2):
    b = pl.program_id(0)
    n = pl.program_id(1)
    x1 = x1_ref[0]
    x2 = x2_ref[0]
    a0, a1, a2 = x1[:, 0:1], x1[:, 1:2], x1[:, 2:3]
    b0, b1, b2 = x2[0:1, :], x2[1:2, :], x2[2:3, :]
    s1 = a0 * a0 + a1 * a1 + a2 * a2
    s2 = b0 * b0 + b1 * b1 + b2 * b2
    xy = lax.dot_general(x1.astype(jnp.bfloat16), x2.astype(jnp.bfloat16),
                         (((1,), (0,)), ((), ())),
                         preferred_element_type=jnp.float32)
    d2 = s1 + s2 - 2.0 * xy

    iota = lax.broadcasted_iota(jnp.int32, (tq, n2), 1)
    dcur = d2
    ws = []
    idxs = []
    for k in range(3):
        m = jnp.min(dcur, axis=1, keepdims=True)
        ik = jnp.argmin(dcur, axis=1).reshape(tq, 1).astype(jnp.int32)
        dist = jnp.sqrt(jnp.maximum(m, 1e-12))
        ws.append(1.0 / (dist + 1e-8))
        idxs.append(ik)
        if k < 2:
            dcur = jnp.where(iota == ik, jnp.inf, dcur)
    wsum = ws[0] + ws[1] + ws[2]

    s_mat = jnp.where(iota == idxs[0], ws[0] / wsum, 0.0)
    s_mat = s_mat + jnp.where(iota == idxs[1], ws[1] / wsum, 0.0)
    s_mat = s_mat + jnp.where(iota == idxs[2], ws[2] / wsum, 0.0)

    f2 = f2_ref[0]
    interp = lax.dot_general(f2, s_mat, (((1,), (1,)), ((), ())))
    y = (jnp.dot(wa_ref[...], interp) +
         jnp.dot(wb_ref[...], f1_ref[0]))
    y_ref[0] = y

    @pl.when((b == 0) & (n == 0))
    def _init():
        ss_ref[...] = jnp.zeros_like(ss_ref)

    part = jnp.concatenate(
        [jnp.sum(y, axis=1, keepdims=True),
         jnp.sum(y * y, axis=1, keepdims=True)], axis=1)
    ss_ref[...] += part


def _bn_body(y_ref, ss_ref, g_ref, bta_ref, o_ref, *, count):
    s = ss_ref[...]
    mean = s[:, 0:1] / count
    var = s[:, 1:2] / count - mean * mean
    scale = g_ref[...] / jnp.sqrt(var + 1e-5)
    shift = bta_ref[...] - mean * scale
    y = y_ref[0]
    o_ref[0] = jnp.maximum(y * scale + shift, 0.0)


def kernel(xyz1, xyz2, feats1, feats2, W, gamma, beta):
    B, N1, _ = xyz1.shape
    N2 = xyz2.shape[1]
    C1 = feats1.shape[1]
    C2 = feats2.shape[1]
    OUT = W.shape[0]
    TQ = 512 if N1 % 512 == 0 else N1

    x1p = jnp.pad(xyz1, ((0, 0), (0, 0), (0, 5)))
    x2p = jnp.pad(jnp.transpose(xyz2, (0, 2, 1)), ((0, 0), (0, 5), (0, 0)))
    Wa = W[:, :C2]
    Wb = W[:, C2:]

    y, ss = pl.pallas_call(
        functools.partial(_main_body, tq=TQ, n2=N2),
        grid=(B, N1 // TQ),
        in_specs=[
            pl.BlockSpec((1, TQ, 8), lambda b, n: (b, n, 0)),
            pl.BlockSpec((1, 8, N2), lambda b, n: (b, 0, 0)),
            pl.BlockSpec((1, C1, TQ), lambda b, n: (b, 0, n)),
            pl.BlockSpec((1, C2, N2), lambda b, n: (b, 0, 0)),
            pl.BlockSpec((OUT, C2), lambda b, n: (0, 0)),
            pl.BlockSpec((OUT, C1), lambda b, n: (0, 0)),
        ],
        out_specs=[
            pl.BlockSpec((1, OUT, TQ), lambda b, n: (b, 0, n)),
            pl.BlockSpec((OUT, 2), lambda b, n: (0, 0)),
        ],
        out_shape=[
            jax.ShapeDtypeStruct((B, OUT, N1), jnp.float32),
            jax.ShapeDtypeStruct((OUT, 2), jnp.float32),
        ],
    )(x1p, x2p, feats1, feats2, Wa, Wb)

    TN = 1024 if N1 % 1024 == 0 else N1
    out = pl.pallas_call(
        functools.partial(_bn_body, count=float(B * N1)),
        grid=(B, N1 // TN),
        in_specs=[
            pl.BlockSpec((1, OUT, TN), lambda b, n: (b, 0, n)),
            pl.BlockSpec((OUT, 2), lambda b, n: (0, 0)),
            pl.BlockSpec((OUT, 1), lambda b, n: (0, 0)),
            pl.BlockSpec((OUT, 1), lambda b, n: (0, 0)),
        ],
        out_specs=pl.BlockSpec((1, OUT, TN), lambda b, n: (b, 0, n)),
        out_shape=jax.ShapeDtypeStruct((B, OUT, N1), jnp.float32),
    )(y, ss, gamma.reshape(OUT, 1), beta.reshape(OUT, 1))
    return out
